# Initial kernel scaffold; baseline (speedup 1.0000x reference)
import jax
import jax.numpy as jnp
from jax.experimental import pallas as pl


def kernel(x, edge_index, W1, b1, W2, b2):
    raise NotImplementedError("write your pallas kernel here")


# speedup vs baseline: 18.2017x; 18.2017x over previous
import jax
import jax.numpy as jnp
from jax import lax
from jax.experimental import pallas as pl
from jax.experimental.pallas import tpu as pltpu
from jax.experimental.pallas import tpu_sc as plsc

N = 10000
E = 320000
D = 128
DH = D // 2
NC = 2
NS = 16
EPT = E // NS
CHUNK = 100
NCH = EPT // CHUNK
NCHP = NCH + 2
NP = 10240
RPT = NP // NS
BLK = 2000

_mesh = plsc.VectorSubcoreMesh(
    core_axis_name="c", subcore_axis_name="s", num_cores=NC, num_subcores=NS)
_sc_params = pltpu.CompilerParams(use_tc_tiling_on_sc=False)


def _deg_body(dst_hbm, zeros16_hbm, ones_hbm, out_hbm, dst_v, ones_v, acc_sh):
    c = lax.axis_index("c")
    s = lax.axis_index("s")
    pltpu.sync_copy(dst_hbm.at[s], dst_v)
    pltpu.sync_copy(ones_hbm, ones_v)
    pltpu.sync_copy(zeros16_hbm.at[pl.ds(s * RPT, RPT)],
                    acc_sh.at[pl.ds(s * RPT, RPT)])
    plsc.subcore_barrier()

    def body(j, carry):
        pltpu.sync_copy(ones_v, acc_sh.at[dst_v.at[j]], add=True)
        return carry

    lax.fori_loop(0, NCH, body, 0)
    plsc.subcore_barrier()
    pltpu.sync_copy(acc_sh.at[pl.ds(s * RPT, RPT)],
                    out_hbm.at[pl.ds(c * NP + s * RPT, RPT)])


_deg_call = pl.kernel(
    _deg_body,
    out_type=jax.ShapeDtypeStruct((NC * NP, 16), jnp.float32),
    mesh=_mesh,
    scratch_types=[
        pltpu.VMEM((NCHP, CHUNK), jnp.int32),
        pltpu.VMEM((CHUNK, 16), jnp.float32),
        pltpu.VMEM_SHARED((NP, 16), jnp.float32),
    ],
    compiler_params=_sc_params,
)


def _edge_body(src_hbm, dst_hbm, g_hbm, zeros_hbm, out_hbm,
               src_v, dst_v, buf0, buf1, acc_sh, sem0, sem1):
    c = lax.axis_index("c")
    s = lax.axis_index("s")
    pltpu.sync_copy(src_hbm.at[c * NS + s], src_v)
    pltpu.sync_copy(dst_hbm.at[s], dst_v)
    pltpu.sync_copy(zeros_hbm.at[pl.ds(s * RPT, RPT)],
                    acc_sh.at[pl.ds(s * RPT, RPT)])
    plsc.subcore_barrier()

    pltpu.async_copy(g_hbm.at[src_v.at[0]], buf0, sem0)
    pltpu.async_copy(g_hbm.at[src_v.at[1]], buf1, sem1)

    def body(j2, carry):
        j = 2 * j2
        pltpu.make_async_copy(g_hbm.at[src_v.at[j]], buf0, sem0).wait()
        pltpu.sync_copy(buf0, acc_sh.at[dst_v.at[j]], add=True)
        pltpu.async_copy(g_hbm.at[src_v.at[j + 2]], buf0, sem0)
        pltpu.make_async_copy(g_hbm.at[src_v.at[j + 1]], buf1, sem1).wait()
        pltpu.sync_copy(buf1, acc_sh.at[dst_v.at[j + 1]], add=True)
        pltpu.async_copy(g_hbm.at[src_v.at[j + 3]], buf1, sem1)
        return carry

    lax.fori_loop(0, NCH // 2, body, 0)
    pltpu.make_async_copy(g_hbm.at[src_v.at[0]], buf0, sem0).wait()
    pltpu.make_async_copy(g_hbm.at[src_v.at[1]], buf1, sem1).wait()
    plsc.subcore_barrier()
    pltpu.sync_copy(acc_sh.at[pl.ds(s * RPT, RPT)],
                    out_hbm.at[pl.ds(c * NP + s * RPT, RPT)])


_edge_call = pl.kernel(
    _edge_body,
    out_type=jax.ShapeDtypeStruct((NC * NP, DH), jnp.float32),
    mesh=_mesh,
    scratch_types=[
        pltpu.VMEM((NCHP, CHUNK), jnp.int32),
        pltpu.VMEM((NCHP, CHUNK), jnp.int32),
        pltpu.VMEM((CHUNK, DH), jnp.float32),
        pltpu.VMEM((CHUNK, DH), jnp.float32),
        pltpu.VMEM_SHARED((NP, DH), jnp.float32),
        pltpu.SemaphoreType.DMA,
        pltpu.SemaphoreType.DMA,
    ],
    compiler_params=_sc_params,
)


def _dinv(d_ref):
    return lax.rsqrt(d_ref[...][:, 0:1] + 1.0)


def _split_store(g_ref, res):
    g_ref[0, :, :] = res[:, :DH]
    g_ref[1, :, :] = res[:, DH:]


def _prep_tc(x_ref, w_ref, d_ref, g_ref):
    dinv = _dinv(d_ref)
    _split_store(g_ref, dinv * jnp.dot(x_ref[...], w_ref[...],
                                       preferred_element_type=jnp.float32))


def _mid_tc(aL_ref, aR_ref, g1_ref, d_ref, b_ref, w_ref, g2_ref):
    dinv = _dinv(d_ref)
    acc = jnp.concatenate([aL_ref[...], aR_ref[...]], axis=1)
    g1 = jnp.concatenate([g1_ref[0, :, :], g1_ref[1, :, :]], axis=1)
    z = jnp.maximum(dinv * (acc + g1) + b_ref[...], 0.0)
    _split_store(g2_ref, dinv * jnp.dot(z, w_ref[...],
                                        preferred_element_type=jnp.float32))


def _out_tc(aL_ref, aR_ref, g2_ref, d_ref, b_ref, out_ref):
    dinv = _dinv(d_ref)
    acc = jnp.concatenate([aL_ref[...], aR_ref[...]], axis=1)
    g2 = jnp.concatenate([g2_ref[0, :, :], g2_ref[1, :, :]], axis=1)
    out_ref[...] = jnp.maximum(dinv * (acc + g2) + b_ref[...], 0.0)


_row_spec = pl.BlockSpec((BLK, D), lambda i: (i, 0))
_half_spec = pl.BlockSpec((BLK, DH), lambda i: (i, 0))
_pair_spec = pl.BlockSpec((2, BLK, DH), lambda i: (0, i, 0))
_deg_spec = pl.BlockSpec((BLK, 16), lambda i: (i, 0))
_mat_spec = pl.BlockSpec((D, D), lambda i: (0, 0))
_bias_spec = pl.BlockSpec((1, D), lambda i: (0, 0))
_grid = (N // BLK,)
_pair_shape = jax.ShapeDtypeStruct((2, N, DH), jnp.float32)

_prep_call = pl.pallas_call(
    _prep_tc, grid=_grid,
    in_specs=[_row_spec, _mat_spec, _deg_spec],
    out_specs=_pair_spec,
    out_shape=_pair_shape)

_mid_call = pl.pallas_call(
    _mid_tc, grid=_grid,
    in_specs=[_half_spec, _half_spec, _pair_spec, _deg_spec,
              _bias_spec, _mat_spec],
    out_specs=_pair_spec,
    out_shape=_pair_shape)

_out_call = pl.pallas_call(
    _out_tc, grid=_grid,
    in_specs=[_half_spec, _half_spec, _pair_spec, _deg_spec, _bias_spec],
    out_specs=_row_spec,
    out_shape=jax.ShapeDtypeStruct((N, D), jnp.float32))


def kernel(x, edge_index, W1, b1, W2, b2):
    ei = edge_index.astype(jnp.int32)
    src = ei[0].reshape(NS, NCH, CHUNK)
    dst = ei[1].reshape(NS, NCH, CHUNK)
    pad = jnp.zeros((NS, NCHP - NCH, CHUNK), jnp.int32)
    src_p = jnp.concatenate([src, pad], axis=1)
    dst_p = jnp.concatenate([dst, pad], axis=1)
    src_all = jnp.concatenate([src_p, src_p + N], axis=0)
    zeros_dh = jnp.zeros((NP, DH), jnp.float32)
    zeros16 = jnp.zeros((NP, 16), jnp.float32)
    ones16 = jnp.ones((CHUNK, 16), jnp.float32)
    b1r = b1.reshape(1, D)
    b2r = b2.reshape(1, D)

    degp = _deg_call(dst_p, zeros16, ones16)
    d0 = degp[:N]

    g1 = _prep_call(x, W1, d0)
    acc1 = _edge_call(src_all, dst_p, g1.reshape(2 * N, DH), zeros_dh)
    g2 = _mid_call(acc1[:N], acc1[NP:NP + N], g1, d0, b1r, W2)
    acc2 = _edge_call(src_all, dst_p, g2.reshape(2 * N, DH), zeros_dh)
    return _out_call(acc2[:N], acc2[NP:NP + N], g2, d0, b2r)
